# baseline (speedup 1.0000x reference)
import functools

import jax
import jax.numpy as jnp
from jax import lax
from jax.experimental import pallas as pl
from jax.experimental.pallas import tpu as pltpu
from jax.experimental.pallas import tpu_sc as plsc

_LN2 = 0.6931471805599453
_C3 = 1.0668396110e-01
_C2 = -3.9353356129e-01
_C1 = 2.8660465269e-01
_C0 = 9.2530396686e-04


def _vlog(v):
    bits = lax.bitcast_convert_type(v, jnp.int32)
    g = bits.astype(jnp.float32) * jnp.float32(2.0**-23) - jnp.float32(127.0)
    m = lax.bitcast_convert_type(
        jnp.bitwise_or(jnp.bitwise_and(bits, 0x007FFFFF), 0x3F800000),
        jnp.float32,
    )
    t = m - jnp.float32(1.0)
    p = (jnp.float32(_C3) * t + jnp.float32(_C2)) * t + jnp.float32(_C1)
    return jnp.float32(_LN2) * g + (p * t + jnp.float32(_C0))


@functools.partial(jax.jit, static_argnames=("unroll",))
def _gather_log_cols(x_t, probs_t, unroll=16):
    d, v = probs_t.shape
    h_len, b_len = x_t.shape
    info = plsc.get_sparse_core_info()
    nc, ns = info.num_cores, info.num_subcores
    s_per_c = d // nc
    nh = -(-h_len // ns)
    blk = nh * b_len

    mesh = plsc.VectorSubcoreMesh(core_axis_name="c", subcore_axis_name="s")

    @functools.partial(
        pl.kernel,
        mesh=mesh,
        out_type=jax.ShapeDtypeStruct((h_len, d, b_len), jnp.float32),
        scratch_types=[
            pltpu.VMEM((b_len,), jnp.int32),
            pltpu.VMEM((b_len,), jnp.int32),
            pltpu.VMEM((b_len,), jnp.int32),
            pltpu.VMEM((blk,), jnp.float32),
            pltpu.VMEM_SHARED((v,), jnp.float32),
            pltpu.SemaphoreType.DMA,
            pltpu.SemaphoreType.DMA,
            pltpu.SemaphoreType.DMA,
            pltpu.SemaphoreType.DMA,
            pltpu.SemaphoreType.DMA,
            pltpu.SemaphoreType.DMA,
            pltpu.SemaphoreType.DMA,
        ],
        compiler_params=pltpu.CompilerParams(
            needs_layout_passes=False, use_tc_tiling_on_sc=True
        ),
    )
    def body(
        x_hbm, probs_hbm, out_hbm,
        i0, i1, i2, buf_v, col_sh,
        s0, s1, s2, g0, g1, wsem, csem,
    ):
        idx = (i0, i1, i2)
        isem = (s0, s1, s2)
        gsem = (g0, g1)
        c = lax.axis_index("c")
        t = lax.axis_index("s")
        h0 = jnp.minimum(t * nh, h_len - nh)

        def idxload(h_rel):
            return pltpu.async_copy(
                x_hbm.at[h0 + h_rel, :], idx[h_rel % 3], isem[h_rel % 3]
            )

        def gfire(h_rel):
            return pltpu.async_copy(
                col_sh.at[idx[h_rel % 3]],
                buf_v.at[pl.ds(h_rel * b_len, b_len)],
                gsem[h_rel % 2],
            )

        def compute_row(h_rel, s_abs):
            def lbody(i, carry2, base=h_rel * b_len):
                for u in range(unroll):
                    p = base + (i * unroll + u) * 16
                    buf_v[pl.ds(p, 16)] = _vlog(buf_v[pl.ds(p, 16)])
                return carry2

            lax.fori_loop(0, b_len // (16 * unroll), lbody, 0)
            return pltpu.async_copy(
                buf_v.at[pl.ds(h_rel * b_len, b_len)],
                out_hbm.at[h0 + h_rel, s_abs, :],
                wsem,
            )

        @pl.when(t == 0)
        def _():
            pltpu.async_copy(probs_hbm.at[c * s_per_c], col_sh, csem)

        def state_body(k, carry):
            s_abs = c * s_per_c + k
            il = [idxload(0), idxload(1), idxload(2)] + [None] * (nh - 3)

            @pl.when(t == 0)
            def _():
                pltpu.make_async_copy(probs_hbm.at[s_abs], col_sh, csem).wait()

            plsc.subcore_barrier()

            gh = [None] * nh
            writes = [None] * nh
            for h_rel in range(nh):
                il[h_rel].wait()
                gh[h_rel] = gfire(h_rel)
                if h_rel >= 1:
                    gh[h_rel - 1].wait()
                    if h_rel + 2 < nh:
                        il[h_rel + 2] = idxload(h_rel + 2)
                    writes[h_rel - 1] = compute_row(h_rel - 1, s_abs)
            gh[nh - 1].wait()
            plsc.subcore_barrier()

            @pl.when((t == 0) & (k + 1 < s_per_c))
            def _():
                pltpu.async_copy(probs_hbm.at[s_abs + 1], col_sh, csem)

            writes[nh - 1] = compute_row(nh - 1, s_abs)
            for w in writes:
                w.wait()
            return carry

        lax.fori_loop(0, s_per_c, state_body, 0)

    return body(x_t, probs_t)


def kernel(x, probs):
    out = _gather_log_cols(x.T.astype(jnp.int32), probs.T)
    return out.transpose(2, 0, 1)


# speedup vs baseline: 1.0139x; 1.0139x over previous
import functools

import jax
import jax.numpy as jnp
from jax import lax
from jax.experimental import pallas as pl
from jax.experimental.pallas import tpu as pltpu
from jax.experimental.pallas import tpu_sc as plsc

_LN2 = 0.6931471805599453
_C3 = 1.0668396110e-01
_C2 = -3.9353356129e-01
_C1 = 2.8660465269e-01
_C0 = 9.2530396686e-04


def _vlog(v):
    bits = lax.bitcast_convert_type(v, jnp.int32)
    g = bits.astype(jnp.float32) * jnp.float32(2.0**-23) - jnp.float32(127.0)
    m = lax.bitcast_convert_type(
        jnp.bitwise_or(jnp.bitwise_and(bits, 0x007FFFFF), 0x3F800000),
        jnp.float32,
    )
    t = m - jnp.float32(1.0)
    p = (jnp.float32(_C3) * t + jnp.float32(_C2)) * t + jnp.float32(_C1)
    return jnp.float32(_LN2) * g + (p * t + jnp.float32(_C0))


@functools.partial(jax.jit, static_argnames=("unroll",))
def _gather_log_cols(x_t, probs_t, unroll=8):
    d, v = probs_t.shape
    h_len, b_len = x_t.shape
    info = plsc.get_sparse_core_info()
    nc, ns = info.num_cores, info.num_subcores
    s_per_c = d // nc
    nh = -(-h_len // ns)
    blk = nh * b_len

    mesh = plsc.VectorSubcoreMesh(core_axis_name="c", subcore_axis_name="s")

    @functools.partial(
        pl.kernel,
        mesh=mesh,
        out_type=jax.ShapeDtypeStruct((h_len, d, b_len), jnp.float32),
        scratch_types=[
            pltpu.VMEM((b_len,), jnp.int32),
            pltpu.VMEM((b_len,), jnp.int32),
            pltpu.VMEM((b_len,), jnp.int32),
            pltpu.VMEM((blk,), jnp.float32),
            pltpu.VMEM_SHARED((v,), jnp.float32),
            pltpu.SemaphoreType.DMA,
            pltpu.SemaphoreType.DMA,
            pltpu.SemaphoreType.DMA,
            pltpu.SemaphoreType.DMA,
            pltpu.SemaphoreType.DMA,
            pltpu.SemaphoreType.DMA,
            pltpu.SemaphoreType.DMA,
        ],
        compiler_params=pltpu.CompilerParams(
            needs_layout_passes=False, use_tc_tiling_on_sc=True
        ),
    )
    def body(
        x_hbm, probs_hbm, out_hbm,
        i0, i1, i2, buf_v, col_sh,
        s0, s1, s2, g0, g1, wsem, csem,
    ):
        idx = (i0, i1, i2)
        isem = (s0, s1, s2)
        gsem = (g0, g1)
        c = lax.axis_index("c")
        t = lax.axis_index("s")
        h0 = jnp.minimum(t * nh, h_len - nh)

        def idxload(h_rel):
            return pltpu.async_copy(
                x_hbm.at[h0 + h_rel, :], idx[h_rel % 3], isem[h_rel % 3]
            )

        def gfire(h_rel):
            return pltpu.async_copy(
                col_sh.at[idx[h_rel % 3]],
                buf_v.at[pl.ds(h_rel * b_len, b_len)],
                gsem[h_rel % 2],
            )

        def compute_row(h_rel, s_abs):
            def lbody(i, carry2, base=h_rel * b_len):
                for u in range(unroll):
                    p = base + (i * unroll + u) * 16
                    buf_v[pl.ds(p, 16)] = _vlog(buf_v[pl.ds(p, 16)])
                return carry2

            lax.fori_loop(0, b_len // (16 * unroll), lbody, 0)
            return pltpu.async_copy(
                buf_v.at[pl.ds(h_rel * b_len, b_len)],
                out_hbm.at[h0 + h_rel, s_abs, :],
                wsem,
            )

        @pl.when(t == 0)
        def _():
            pltpu.async_copy(probs_hbm.at[c * s_per_c], col_sh, csem)

        def state_body(k, carry):
            s_abs = c * s_per_c + k
            il = [idxload(0), idxload(1), idxload(2)] + [None] * (nh - 3)

            @pl.when(t == 0)
            def _():
                pltpu.make_async_copy(probs_hbm.at[s_abs], col_sh, csem).wait()

            plsc.subcore_barrier()

            gh = [None] * nh
            writes = [None] * nh
            for h_rel in range(nh):
                il[h_rel].wait()
                gh[h_rel] = gfire(h_rel)
                if h_rel >= 1:
                    gh[h_rel - 1].wait()
                    if h_rel + 2 < nh:
                        il[h_rel + 2] = idxload(h_rel + 2)
                    writes[h_rel - 1] = compute_row(h_rel - 1, s_abs)
            gh[nh - 1].wait()
            plsc.subcore_barrier()

            @pl.when((t == 0) & (k + 1 < s_per_c))
            def _():
                pltpu.async_copy(probs_hbm.at[s_abs + 1], col_sh, csem)

            writes[nh - 1] = compute_row(nh - 1, s_abs)
            for w in writes:
                w.wait()
            return carry

        lax.fori_loop(0, s_per_c, state_body, 0)

    return body(x_t, probs_t)


def kernel(x, probs):
    out = _gather_log_cols(x.T.astype(jnp.int32), probs.T)
    return out.transpose(2, 0, 1)
